# baseline (speedup 1.0000x reference)
import functools

import jax
import jax.numpy as jnp
from jax import lax
from jax.experimental import pallas as pl
from jax.experimental.pallas import tpu as pltpu
from jax.experimental.pallas import tpu_sc as plsc

N_EXPERTS = 8
TOP_K = 2
N_EMBD = 768
HIDDEN = 2048
N_TOKENS = 2048
T_TILE = 128
N_GRP = N_TOKENS // T_TILE
N_ENTRY_ROWS = TOP_K * N_GRP
F_TILE = 256
MAX_TILES = 24
MAX_ROWS = MAX_TILES * F_TILE


def _router_meta_body(x_ref, rw_ref, pos_ref, w_ref, te_ref, nact_ref,
                      laux_ref, zloss_ref):
    x = x_ref[...]
    rw = rw_ref[...]
    logits = jnp.dot(x, rw, preferred_element_type=jnp.float32)
    m = jnp.max(logits, axis=-1, keepdims=True)
    ex = jnp.exp(logits - m)
    se = jnp.sum(ex, axis=-1, keepdims=True)
    probs = ex / se

    cols = lax.broadcasted_iota(jnp.int32, probs.shape, 1)
    w1 = jnp.max(probs, axis=-1, keepdims=True)
    i1 = jnp.argmax(probs, axis=-1)
    is1 = cols == i1[:, None]
    probs2 = jnp.where(is1, -jnp.inf, probs)
    w2 = jnp.max(probs2, axis=-1, keepdims=True)
    i2 = jnp.argmax(probs2, axis=-1)
    denom = w1 + w2 + 1e-9

    n = jnp.float32(N_TOKENS)
    load = jnp.sum(is1.astype(jnp.float32), axis=0) / n
    importance = jnp.mean(probs, axis=0)
    laux_ref[0, 0] = N_EXPERTS * jnp.sum(load * importance)
    lse = m[:, 0] + jnp.log(se[:, 0])
    zloss_ref[0, 0] = jnp.mean(lse * lse)

    e1r = jnp.reshape(i1, (N_GRP, T_TILE))
    e2r = jnp.reshape(i2, (N_GRP, T_TILE))
    w1r = jnp.reshape(w1[:, 0] / denom[:, 0], (N_GRP, T_TILE))
    w2r = jnp.reshape(w2[:, 0] / denom[:, 0], (N_GRP, T_TILE))
    er = jnp.concatenate([e1r, e2r], axis=0)
    w_ref[...] = jnp.concatenate([w1r, w2r], axis=0)

    ra = lax.broadcasted_iota(jnp.int32, (T_TILE, T_TILE), 0)
    ca = lax.broadcasted_iota(jnp.int32, (T_TILE, T_TILE), 1)
    U = (ra < ca).astype(jnp.float32)
    rg = lax.broadcasted_iota(jnp.int32, (N_ENTRY_ROWS, N_ENTRY_ROWS), 0)
    cg = lax.broadcasted_iota(jnp.int32, (N_ENTRY_ROWS, N_ENTRY_ROWS), 1)
    Lg = (rg > cg).astype(jnp.float32)

    inds, ranks, counts = [], [], []
    for e in range(N_EXPERTS):
        ind = (er == e).astype(jnp.float32)
        rank_in = jnp.dot(ind, U, preferred_element_type=jnp.float32)
        tot = jnp.sum(ind, axis=1, keepdims=True)
        grp = jnp.dot(Lg, tot, preferred_element_type=jnp.float32)
        inds.append(ind)
        ranks.append(rank_in + grp)
        counts.append(jnp.sum(tot, axis=0, keepdims=True))

    start = jnp.zeros((1, 1), jnp.int32)
    pos_f = jnp.zeros((N_ENTRY_ROWS, T_TILE), jnp.float32)
    tile_starts = []
    for e in range(N_EXPERTS):
        ci = counts[e].astype(jnp.int32)
        pc = ((ci + (F_TILE - 1)) // F_TILE) * F_TILE
        tile_starts.append(start // F_TILE)
        pos_f = pos_f + inds[e] * (start.astype(jnp.float32) + ranks[e])
        start = start + pc
    pos_ref[...] = pos_f.astype(jnp.int32)
    nact_ref[0, 0] = (start // F_TILE)[0, 0]

    tio = lax.broadcasted_iota(jnp.int32, (1, T_TILE), 1)
    te = jnp.zeros((1, T_TILE), jnp.int32)
    for e in range(N_EXPERTS):
        te = te + (tio >= tile_starts[e]).astype(jnp.int32)
    te_ref[...] = te - 1


def _dispatch_body(x_hbm, pos_hbm, xs_hbm, idx_v, rows_v, sem):
    wid = lax.axis_index("s") * 2 + lax.axis_index("c")
    tok0 = (wid % N_GRP) * T_TILE
    pltpu.sync_copy(pos_hbm.at[wid], idx_v)
    pltpu.sync_copy(x_hbm.at[pl.ds(tok0, T_TILE)], rows_v)
    pltpu.async_copy(rows_v, xs_hbm.at[idx_v], sem).wait()


def _ffn_body(te_ref, nact_ref, xs_ref, wfc_ref, wproj_ref, y_ref):
    t = pl.program_id(0)

    @pl.when(t < nact_ref[0])
    def _():
        x = xs_ref[...]
        h = jnp.dot(x, wfc_ref[0], preferred_element_type=jnp.float32)
        gate = h[:, :HIDDEN]
        val = h[:, HIDDEN:]
        act = gate * jax.nn.sigmoid(gate) * val
        y_ref[...] = jnp.dot(act, wproj_ref[0],
                             preferred_element_type=jnp.float32)


def _combine_body(ys_hbm, pos_hbm, w_hbm, y_hbm,
                  idx1_v, idx2_v, w1_v, w2_v, r1_v, r2_v, sem1, sem2):
    wid = lax.axis_index("s") * 2 + lax.axis_index("c")
    tpw = N_TOKENS // 32
    g = wid // 2
    off = (wid % 2) * tpw
    pltpu.sync_copy(pos_hbm.at[g, pl.ds(off, tpw)], idx1_v)
    pltpu.sync_copy(pos_hbm.at[N_GRP + g, pl.ds(off, tpw)], idx2_v)
    pltpu.sync_copy(w_hbm.at[g, pl.ds(off, tpw)], w1_v)
    pltpu.sync_copy(w_hbm.at[N_GRP + g, pl.ds(off, tpw)], w2_v)
    cp1 = pltpu.async_copy(ys_hbm.at[idx1_v], r1_v, sem1)
    cp2 = pltpu.async_copy(ys_hbm.at[idx2_v], r2_v, sem2)
    cp1.wait()
    cp2.wait()

    dnums = lax.GatherDimensionNumbers(
        offset_dims=(), collapsed_slice_dims=(0,), start_index_map=(0,))
    for jc in range(tpw // 16):
        wch1 = w1_v[pl.ds(jc * 16, 16)]
        wch2 = w2_v[pl.ds(jc * 16, 16)]

        def body(l, _):
            idx = jnp.full((16, 1), l, jnp.int32)
            w1b = lax.gather(wch1, idx, dnums, (1,),
                             mode=lax.GatherScatterMode.PROMISE_IN_BOUNDS)
            w2b = lax.gather(wch2, idx, dnums, (1,),
                             mode=lax.GatherScatterMode.PROMISE_IN_BOUNDS)
            j = jc * 16 + l
            for cc in range(N_EMBD // 16):
                sl = pl.ds(cc * 16, 16)
                r1_v[j, sl] = r1_v[j, sl] * w1b + r2_v[j, sl] * w2b
            return 0

        lax.fori_loop(0, 16, body, 0)
    pltpu.sync_copy(r1_v, y_hbm.at[pl.ds(wid * tpw, tpw)])


def kernel(x, router_W, W_fc, W_proj):
    B, T, C = x.shape
    N = B * T
    x_flat = x.reshape(N, C)

    pos, w, te, nact, laux, zloss = pl.pallas_call(
        _router_meta_body,
        out_shape=(
            jax.ShapeDtypeStruct((N_ENTRY_ROWS, T_TILE), jnp.int32),
            jax.ShapeDtypeStruct((N_ENTRY_ROWS, T_TILE), jnp.float32),
            jax.ShapeDtypeStruct((1, T_TILE), jnp.int32),
            jax.ShapeDtypeStruct((1, 1), jnp.int32),
            jax.ShapeDtypeStruct((1, 1), jnp.float32),
            jax.ShapeDtypeStruct((1, 1), jnp.float32),
        ),
        in_specs=[
            pl.BlockSpec(memory_space=pltpu.VMEM),
            pl.BlockSpec(memory_space=pltpu.VMEM),
        ],
        out_specs=(
            pl.BlockSpec(memory_space=pltpu.VMEM),
            pl.BlockSpec(memory_space=pltpu.VMEM),
            pl.BlockSpec(memory_space=pltpu.VMEM),
            pl.BlockSpec(memory_space=pltpu.SMEM),
            pl.BlockSpec(memory_space=pltpu.SMEM),
            pl.BlockSpec(memory_space=pltpu.SMEM),
        ),
    )(x_flat, router_W)

    mesh = plsc.VectorSubcoreMesh(core_axis_name="c", subcore_axis_name="s")

    dispatch = pl.kernel(
        _dispatch_body,
        mesh=mesh,
        out_type=jax.ShapeDtypeStruct((MAX_ROWS, C), jnp.float32),
        scratch_types=[
            pltpu.VMEM((T_TILE,), jnp.int32),
            pltpu.VMEM((T_TILE, C), jnp.float32),
            pltpu.SemaphoreType.DMA,
        ],
    )
    xs = dispatch(x_flat, pos)

    grid_spec = pltpu.PrefetchScalarGridSpec(
        num_scalar_prefetch=2,
        grid=(MAX_TILES,),
        in_specs=[
            pl.BlockSpec((F_TILE, C), lambda t, te_r, na_r: (t, 0)),
            pl.BlockSpec((1, C, 2 * HIDDEN),
                         lambda t, te_r, na_r: (te_r[t], 0, 0)),
            pl.BlockSpec((1, HIDDEN, C),
                         lambda t, te_r, na_r: (te_r[t], 0, 0)),
        ],
        out_specs=pl.BlockSpec((F_TILE, C), lambda t, te_r, na_r: (t, 0)),
    )
    ys = pl.pallas_call(
        _ffn_body,
        grid_spec=grid_spec,
        out_shape=jax.ShapeDtypeStruct((MAX_ROWS, C), jnp.float32),
    )(te[0], nact.reshape((1,)), xs, W_fc, W_proj)

    tpw = N_TOKENS // 32
    combine = pl.kernel(
        _combine_body,
        mesh=mesh,
        out_type=jax.ShapeDtypeStruct((N, C), jnp.float32),
        scratch_types=[
            pltpu.VMEM((tpw,), jnp.int32),
            pltpu.VMEM((tpw,), jnp.int32),
            pltpu.VMEM((tpw,), jnp.float32),
            pltpu.VMEM((tpw,), jnp.float32),
            pltpu.VMEM((tpw, C), jnp.float32),
            pltpu.VMEM((tpw, C), jnp.float32),
            pltpu.SemaphoreType.DMA,
            pltpu.SemaphoreType.DMA,
        ],
    )
    y_flat = combine(ys, pos, w)

    return (y_flat.reshape(B, T, C), laux[0, 0], zloss[0, 0])


# speedup vs baseline: 1.1472x; 1.1472x over previous
import functools

import jax
import jax.numpy as jnp
from jax import lax
from jax.experimental import pallas as pl
from jax.experimental.pallas import tpu as pltpu
from jax.experimental.pallas import tpu_sc as plsc

N_EXPERTS = 8
TOP_K = 2
N_EMBD = 768
HIDDEN = 2048
N_TOKENS = 2048
T_TILE = 128
N_GRP = N_TOKENS // T_TILE
N_ENTRY_ROWS = TOP_K * N_GRP
F_TILE = 768
MAX_TILES = 13
MAX_ROWS = MAX_TILES * F_TILE


def _router_meta_body(x_ref, rw_ref, pos_ref, w_ref, te_ref, nact_ref,
                      laux_ref, zloss_ref):
    x = x_ref[...]
    rw = rw_ref[...]
    logits = jnp.dot(x, rw, preferred_element_type=jnp.float32)
    m = jnp.max(logits, axis=-1, keepdims=True)
    ex = jnp.exp(logits - m)
    se = jnp.sum(ex, axis=-1, keepdims=True)
    probs = ex / se

    cols = lax.broadcasted_iota(jnp.int32, probs.shape, 1)
    w1 = jnp.max(probs, axis=-1, keepdims=True)
    i1 = jnp.argmax(probs, axis=-1)
    is1 = cols == i1[:, None]
    probs2 = jnp.where(is1, -jnp.inf, probs)
    w2 = jnp.max(probs2, axis=-1, keepdims=True)
    i2 = jnp.argmax(probs2, axis=-1)
    denom = w1 + w2 + 1e-9

    n = jnp.float32(N_TOKENS)
    load = jnp.sum(is1.astype(jnp.float32), axis=0) / n
    importance = jnp.mean(probs, axis=0)
    laux_ref[0, 0] = N_EXPERTS * jnp.sum(load * importance)
    lse = m[:, 0] + jnp.log(se[:, 0])
    zloss_ref[0, 0] = jnp.mean(lse * lse)

    e1r = jnp.reshape(i1, (N_GRP, T_TILE))
    e2r = jnp.reshape(i2, (N_GRP, T_TILE))
    w1r = jnp.reshape(w1[:, 0] / denom[:, 0], (N_GRP, T_TILE))
    w2r = jnp.reshape(w2[:, 0] / denom[:, 0], (N_GRP, T_TILE))
    er = jnp.concatenate([e1r, e2r], axis=0)
    w_ref[...] = jnp.concatenate([w1r, w2r], axis=0)

    ra = lax.broadcasted_iota(jnp.int32, (T_TILE, T_TILE), 0)
    ca = lax.broadcasted_iota(jnp.int32, (T_TILE, T_TILE), 1)
    U = (ra < ca).astype(jnp.float32)
    rg = lax.broadcasted_iota(jnp.int32, (N_ENTRY_ROWS, N_ENTRY_ROWS), 0)
    cg = lax.broadcasted_iota(jnp.int32, (N_ENTRY_ROWS, N_ENTRY_ROWS), 1)
    Lg = (rg > cg).astype(jnp.float32)

    inds, ranks, counts = [], [], []
    for e in range(N_EXPERTS):
        ind = (er == e).astype(jnp.float32)
        rank_in = jnp.dot(ind, U, preferred_element_type=jnp.float32)
        tot = jnp.sum(ind, axis=1, keepdims=True)
        grp = jnp.dot(Lg, tot, preferred_element_type=jnp.float32)
        inds.append(ind)
        ranks.append(rank_in + grp)
        counts.append(jnp.sum(tot, axis=0, keepdims=True))

    start = jnp.zeros((1, 1), jnp.int32)
    pos_f = jnp.zeros((N_ENTRY_ROWS, T_TILE), jnp.float32)
    tile_starts = []
    for e in range(N_EXPERTS):
        ci = counts[e].astype(jnp.int32)
        pc = ((ci + (F_TILE - 1)) // F_TILE) * F_TILE
        tile_starts.append(start // F_TILE)
        pos_f = pos_f + inds[e] * (start.astype(jnp.float32) + ranks[e])
        start = start + pc
    pos_ref[...] = pos_f.astype(jnp.int32)
    nact_ref[0, 0] = (start // F_TILE)[0, 0]

    tio = lax.broadcasted_iota(jnp.int32, (1, T_TILE), 1)
    te = jnp.zeros((1, T_TILE), jnp.int32)
    for e in range(N_EXPERTS):
        te = te + (tio >= tile_starts[e]).astype(jnp.int32)
    te_ref[...] = te - 1


def _dispatch_body(x_hbm, pos_hbm, xs_hbm, idx_v, rows_v, sem):
    wid = lax.axis_index("s") * 2 + lax.axis_index("c")
    tok0 = (wid % N_GRP) * T_TILE
    pltpu.sync_copy(pos_hbm.at[wid], idx_v)
    pltpu.sync_copy(x_hbm.at[pl.ds(tok0, T_TILE)], rows_v)
    pltpu.async_copy(rows_v, xs_hbm.at[idx_v], sem).wait()


def _ffn_body(te_ref, nact_ref, xs_ref, wfc_ref, wproj_ref, y_ref):
    t = pl.program_id(0)

    @pl.when(t < nact_ref[0])
    def _():
        x = xs_ref[...]
        h = jnp.dot(x, wfc_ref[0], preferred_element_type=jnp.float32)
        gate = h[:, :HIDDEN]
        val = h[:, HIDDEN:]
        act = gate * jax.nn.sigmoid(gate) * val
        y_ref[...] = jnp.dot(act, wproj_ref[0],
                             preferred_element_type=jnp.float32)


def _combine_body(ys_hbm, pos_hbm, w_hbm, y_hbm,
                  idx1_v, idx2_v, w1_v, w2_v, r1_v, r2_v, sem1, sem2):
    wid = lax.axis_index("s") * 2 + lax.axis_index("c")
    tpw = N_TOKENS // 32
    g = wid // 2
    off = (wid % 2) * tpw
    pltpu.sync_copy(pos_hbm.at[g, pl.ds(off, tpw)], idx1_v)
    pltpu.sync_copy(pos_hbm.at[N_GRP + g, pl.ds(off, tpw)], idx2_v)
    pltpu.sync_copy(w_hbm.at[g, pl.ds(off, tpw)], w1_v)
    pltpu.sync_copy(w_hbm.at[N_GRP + g, pl.ds(off, tpw)], w2_v)
    cp1 = pltpu.async_copy(ys_hbm.at[idx1_v], r1_v, sem1)
    cp2 = pltpu.async_copy(ys_hbm.at[idx2_v], r2_v, sem2)
    cp1.wait()
    cp2.wait()

    dnums = lax.GatherDimensionNumbers(
        offset_dims=(), collapsed_slice_dims=(0,), start_index_map=(0,))
    for jc in range(tpw // 16):
        wch1 = w1_v[pl.ds(jc * 16, 16)]
        wch2 = w2_v[pl.ds(jc * 16, 16)]

        def body(l, _):
            idx = jnp.full((16, 1), l, jnp.int32)
            w1b = lax.gather(wch1, idx, dnums, (1,),
                             mode=lax.GatherScatterMode.PROMISE_IN_BOUNDS)
            w2b = lax.gather(wch2, idx, dnums, (1,),
                             mode=lax.GatherScatterMode.PROMISE_IN_BOUNDS)
            j = jc * 16 + l
            for cc in range(N_EMBD // 16):
                sl = pl.ds(cc * 16, 16)
                r1_v[j, sl] = r1_v[j, sl] * w1b + r2_v[j, sl] * w2b
            return 0

        lax.fori_loop(0, 16, body, 0)
    pltpu.sync_copy(r1_v, y_hbm.at[pl.ds(wid * tpw, tpw)])


def kernel(x, router_W, W_fc, W_proj):
    B, T, C = x.shape
    N = B * T
    x_flat = x.reshape(N, C)

    pos, w, te, nact, laux, zloss = pl.pallas_call(
        _router_meta_body,
        out_shape=(
            jax.ShapeDtypeStruct((N_ENTRY_ROWS, T_TILE), jnp.int32),
            jax.ShapeDtypeStruct((N_ENTRY_ROWS, T_TILE), jnp.float32),
            jax.ShapeDtypeStruct((1, T_TILE), jnp.int32),
            jax.ShapeDtypeStruct((1, 1), jnp.int32),
            jax.ShapeDtypeStruct((1, 1), jnp.float32),
            jax.ShapeDtypeStruct((1, 1), jnp.float32),
        ),
        in_specs=[
            pl.BlockSpec(memory_space=pltpu.VMEM),
            pl.BlockSpec(memory_space=pltpu.VMEM),
        ],
        out_specs=(
            pl.BlockSpec(memory_space=pltpu.VMEM),
            pl.BlockSpec(memory_space=pltpu.VMEM),
            pl.BlockSpec(memory_space=pltpu.VMEM),
            pl.BlockSpec(memory_space=pltpu.SMEM),
            pl.BlockSpec(memory_space=pltpu.SMEM),
            pl.BlockSpec(memory_space=pltpu.SMEM),
        ),
    )(x_flat, router_W)

    mesh = plsc.VectorSubcoreMesh(core_axis_name="c", subcore_axis_name="s")

    dispatch = pl.kernel(
        _dispatch_body,
        mesh=mesh,
        out_type=jax.ShapeDtypeStruct((MAX_ROWS, C), jnp.float32),
        scratch_types=[
            pltpu.VMEM((T_TILE,), jnp.int32),
            pltpu.VMEM((T_TILE, C), jnp.float32),
            pltpu.SemaphoreType.DMA,
        ],
    )
    xs = dispatch(x_flat, pos)

    grid_spec = pltpu.PrefetchScalarGridSpec(
        num_scalar_prefetch=2,
        grid=(MAX_TILES,),
        in_specs=[
            pl.BlockSpec((F_TILE, C), lambda t, te_r, na_r: (t, 0)),
            pl.BlockSpec((1, C, 2 * HIDDEN),
                         lambda t, te_r, na_r: (te_r[t], 0, 0)),
            pl.BlockSpec((1, HIDDEN, C),
                         lambda t, te_r, na_r: (te_r[t], 0, 0)),
        ],
        out_specs=pl.BlockSpec((F_TILE, C), lambda t, te_r, na_r: (t, 0)),
    )
    ys = pl.pallas_call(
        _ffn_body,
        grid_spec=grid_spec,
        out_shape=jax.ShapeDtypeStruct((MAX_ROWS, C), jnp.float32),
    )(te[0], nact.reshape((1,)), xs, W_fc, W_proj)

    tpw = N_TOKENS // 32
    combine = pl.kernel(
        _combine_body,
        mesh=mesh,
        out_type=jax.ShapeDtypeStruct((N, C), jnp.float32),
        scratch_types=[
            pltpu.VMEM((tpw,), jnp.int32),
            pltpu.VMEM((tpw,), jnp.int32),
            pltpu.VMEM((tpw,), jnp.float32),
            pltpu.VMEM((tpw,), jnp.float32),
            pltpu.VMEM((tpw, C), jnp.float32),
            pltpu.VMEM((tpw, C), jnp.float32),
            pltpu.SemaphoreType.DMA,
            pltpu.SemaphoreType.DMA,
        ],
    )
    y_flat = combine(ys, pos, w)

    return (y_flat.reshape(B, T, C), laux[0, 0], zloss[0, 0])


# speedup vs baseline: 1.1521x; 1.0042x over previous
import functools

import jax
import jax.numpy as jnp
from jax import lax
from jax.experimental import pallas as pl
from jax.experimental.pallas import tpu as pltpu
from jax.experimental.pallas import tpu_sc as plsc

N_EXPERTS = 8
TOP_K = 2
N_EMBD = 768
HIDDEN = 2048
N_TOKENS = 2048
T_TILE = 128
N_GRP = N_TOKENS // T_TILE
N_ENTRY_ROWS = TOP_K * N_GRP
F_TILE = 768
MAX_TILES = 13
MAX_ROWS = MAX_TILES * F_TILE


def _router_meta_body(x_ref, rw_ref, pos_ref, w_ref, te_ref, nact_ref,
                      laux_ref, zloss_ref):
    x = x_ref[...]
    rw = rw_ref[...]
    logits = jnp.dot(x, rw, preferred_element_type=jnp.float32)
    m = jnp.max(logits, axis=-1, keepdims=True)
    ex = jnp.exp(logits - m)
    se = jnp.sum(ex, axis=-1, keepdims=True)
    probs = ex / se

    cols = lax.broadcasted_iota(jnp.int32, probs.shape, 1)
    w1 = jnp.max(probs, axis=-1, keepdims=True)
    i1 = jnp.argmax(probs, axis=-1)
    is1 = cols == i1[:, None]
    probs2 = jnp.where(is1, -jnp.inf, probs)
    w2 = jnp.max(probs2, axis=-1, keepdims=True)
    i2 = jnp.argmax(probs2, axis=-1)
    denom = w1 + w2 + 1e-9

    n = jnp.float32(N_TOKENS)
    load = jnp.sum(is1.astype(jnp.float32), axis=0) / n
    importance = jnp.mean(probs, axis=0)
    laux_ref[0, 0] = N_EXPERTS * jnp.sum(load * importance)
    lse = m[:, 0] + jnp.log(se[:, 0])
    zloss_ref[0, 0] = jnp.mean(lse * lse)

    e1r = jnp.reshape(i1, (N_GRP, T_TILE))
    e2r = jnp.reshape(i2, (N_GRP, T_TILE))
    w1r = jnp.reshape(w1[:, 0] / denom[:, 0], (N_GRP, T_TILE))
    w2r = jnp.reshape(w2[:, 0] / denom[:, 0], (N_GRP, T_TILE))
    er = jnp.concatenate([e1r, e2r], axis=0)
    w_ref[...] = jnp.concatenate([w1r, w2r], axis=0)

    ra = lax.broadcasted_iota(jnp.int32, (T_TILE, T_TILE), 0)
    ca = lax.broadcasted_iota(jnp.int32, (T_TILE, T_TILE), 1)
    U = (ra < ca).astype(jnp.float32)
    rg = lax.broadcasted_iota(jnp.int32, (N_ENTRY_ROWS, N_ENTRY_ROWS), 0)
    cg = lax.broadcasted_iota(jnp.int32, (N_ENTRY_ROWS, N_ENTRY_ROWS), 1)
    Lg = (rg > cg).astype(jnp.float32)

    inds, ranks, counts = [], [], []
    for e in range(N_EXPERTS):
        ind = (er == e).astype(jnp.float32)
        rank_in = jnp.dot(ind, U, preferred_element_type=jnp.float32)
        tot = jnp.sum(ind, axis=1, keepdims=True)
        grp = jnp.dot(Lg, tot, preferred_element_type=jnp.float32)
        inds.append(ind)
        ranks.append(rank_in + grp)
        counts.append(jnp.sum(tot, axis=0, keepdims=True))

    start = jnp.zeros((1, 1), jnp.int32)
    pos_f = jnp.zeros((N_ENTRY_ROWS, T_TILE), jnp.float32)
    tile_starts = []
    for e in range(N_EXPERTS):
        ci = counts[e].astype(jnp.int32)
        pc = ((ci + (F_TILE - 1)) // F_TILE) * F_TILE
        tile_starts.append(start // F_TILE)
        pos_f = pos_f + inds[e] * (start.astype(jnp.float32) + ranks[e])
        start = start + pc
    pos_ref[...] = pos_f.astype(jnp.int32)
    nact_ref[0, 0] = (start // F_TILE)[0, 0]

    tio = lax.broadcasted_iota(jnp.int32, (1, T_TILE), 1)
    te = jnp.zeros((1, T_TILE), jnp.int32)
    for e in range(N_EXPERTS):
        te = te + (tio >= tile_starts[e]).astype(jnp.int32)
    te_ref[...] = te - 1


def _dispatch_body(x_hbm, pos_hbm, xs_hbm, idx_v, rows_v, semi, semx, sems):
    wid = lax.axis_index("s") * 2 + lax.axis_index("c")
    tok0 = (wid % N_GRP) * T_TILE
    hh = T_TILE // 2
    ci0 = pltpu.async_copy(pos_hbm.at[wid, pl.ds(0, hh)], idx_v.at[0], semi)
    ci1 = pltpu.async_copy(pos_hbm.at[wid, pl.ds(hh, hh)], idx_v.at[1], semi)
    c0 = pltpu.async_copy(x_hbm.at[pl.ds(tok0, hh)], rows_v.at[pl.ds(0, hh)],
                          semx)
    c1 = pltpu.async_copy(x_hbm.at[pl.ds(tok0 + hh, hh)],
                          rows_v.at[pl.ds(hh, hh)], semx)
    ci0.wait()
    ci1.wait()
    c0.wait()
    s0 = pltpu.async_copy(rows_v.at[pl.ds(0, hh)], xs_hbm.at[idx_v.at[0]],
                          sems)
    c1.wait()
    s1 = pltpu.async_copy(rows_v.at[pl.ds(hh, hh)], xs_hbm.at[idx_v.at[1]],
                          sems)
    s0.wait()
    s1.wait()


def _ffn_body(te_ref, nact_ref, xs_ref, wfc_ref, wproj_ref, y_ref):
    t = pl.program_id(0)

    @pl.when(t < nact_ref[0])
    def _():
        x = xs_ref[...]
        h = jnp.dot(x, wfc_ref[0], preferred_element_type=jnp.float32)
        gate = h[:, :HIDDEN]
        val = h[:, HIDDEN:]
        act = gate * jax.nn.sigmoid(gate) * val
        y_ref[...] = jnp.dot(act, wproj_ref[0],
                             preferred_element_type=jnp.float32)


def _combine_body(ys_hbm, pos_hbm, w_hbm, y_hbm,
                  idx1_v, idx2_v, w1_v, w2_v, r1_v, r2_v,
                  sem1, sem2, sem3, sem4, sem5):
    wid = lax.axis_index("s") * 2 + lax.axis_index("c")
    tpw = N_TOKENS // 32
    hh = tpw // 2
    g = wid // 2
    off = (wid % 2) * tpw
    pltpu.sync_copy(pos_hbm.at[g, pl.ds(off, tpw)], idx1_v)
    pltpu.sync_copy(pos_hbm.at[N_GRP + g, pl.ds(off, tpw)], idx2_v)
    pltpu.sync_copy(w_hbm.at[g, pl.ds(off, tpw)], w1_v)
    pltpu.sync_copy(w_hbm.at[N_GRP + g, pl.ds(off, tpw)], w2_v)
    g1a = pltpu.async_copy(ys_hbm.at[idx1_v.at[pl.ds(0, hh)]],
                           r1_v.at[pl.ds(0, hh)], sem1)
    g2a = pltpu.async_copy(ys_hbm.at[idx2_v.at[pl.ds(0, hh)]],
                           r2_v.at[pl.ds(0, hh)], sem2)
    g1b = pltpu.async_copy(ys_hbm.at[idx1_v.at[pl.ds(hh, hh)]],
                           r1_v.at[pl.ds(hh, hh)], sem3)
    g2b = pltpu.async_copy(ys_hbm.at[idx2_v.at[pl.ds(hh, hh)]],
                           r2_v.at[pl.ds(hh, hh)], sem4)

    dnums = lax.GatherDimensionNumbers(
        offset_dims=(), collapsed_slice_dims=(0,), start_index_map=(0,))

    def mult_quarter(jc):
        wch1 = w1_v[pl.ds(jc * 16, 16)]
        wch2 = w2_v[pl.ds(jc * 16, 16)]

        def body(l, _):
            idx = jnp.full((16, 1), l, jnp.int32)
            w1b = lax.gather(wch1, idx, dnums, (1,),
                             mode=lax.GatherScatterMode.PROMISE_IN_BOUNDS)
            w2b = lax.gather(wch2, idx, dnums, (1,),
                             mode=lax.GatherScatterMode.PROMISE_IN_BOUNDS)
            j = jc * 16 + l
            for cc in range(N_EMBD // 16):
                sl = pl.ds(cc * 16, 16)
                r1_v[j, sl] = r1_v[j, sl] * w1b + r2_v[j, sl] * w2b
            return 0

        lax.fori_loop(0, 16, body, 0)

    g1a.wait()
    g2a.wait()
    mult_quarter(0)
    mult_quarter(1)
    st0 = pltpu.async_copy(r1_v.at[pl.ds(0, hh)],
                           y_hbm.at[pl.ds(wid * tpw, hh)], sem5)
    g1b.wait()
    g2b.wait()
    mult_quarter(2)
    mult_quarter(3)
    st0.wait()
    pltpu.sync_copy(r1_v.at[pl.ds(hh, hh)],
                    y_hbm.at[pl.ds(wid * tpw + hh, hh)])


def kernel(x, router_W, W_fc, W_proj):
    B, T, C = x.shape
    N = B * T
    x_flat = x.reshape(N, C)

    pos, w, te, nact, laux, zloss = pl.pallas_call(
        _router_meta_body,
        out_shape=(
            jax.ShapeDtypeStruct((N_ENTRY_ROWS, T_TILE), jnp.int32),
            jax.ShapeDtypeStruct((N_ENTRY_ROWS, T_TILE), jnp.float32),
            jax.ShapeDtypeStruct((1, T_TILE), jnp.int32),
            jax.ShapeDtypeStruct((1, 1), jnp.int32),
            jax.ShapeDtypeStruct((1, 1), jnp.float32),
            jax.ShapeDtypeStruct((1, 1), jnp.float32),
        ),
        in_specs=[
            pl.BlockSpec(memory_space=pltpu.VMEM),
            pl.BlockSpec(memory_space=pltpu.VMEM),
        ],
        out_specs=(
            pl.BlockSpec(memory_space=pltpu.VMEM),
            pl.BlockSpec(memory_space=pltpu.VMEM),
            pl.BlockSpec(memory_space=pltpu.VMEM),
            pl.BlockSpec(memory_space=pltpu.SMEM),
            pl.BlockSpec(memory_space=pltpu.SMEM),
            pl.BlockSpec(memory_space=pltpu.SMEM),
        ),
    )(x_flat, router_W)

    mesh = plsc.VectorSubcoreMesh(core_axis_name="c", subcore_axis_name="s")

    dispatch = pl.kernel(
        _dispatch_body,
        mesh=mesh,
        out_type=jax.ShapeDtypeStruct((MAX_ROWS, C), jnp.float32),
        scratch_types=[
            pltpu.VMEM((2, T_TILE // 2), jnp.int32),
            pltpu.VMEM((T_TILE, C), jnp.float32),
            pltpu.SemaphoreType.DMA,
            pltpu.SemaphoreType.DMA,
            pltpu.SemaphoreType.DMA,
        ],
    )
    xs = dispatch(x_flat, pos)

    grid_spec = pltpu.PrefetchScalarGridSpec(
        num_scalar_prefetch=2,
        grid=(MAX_TILES,),
        in_specs=[
            pl.BlockSpec((F_TILE, C), lambda t, te_r, na_r: (t, 0)),
            pl.BlockSpec((1, C, 2 * HIDDEN),
                         lambda t, te_r, na_r: (te_r[t], 0, 0)),
            pl.BlockSpec((1, HIDDEN, C),
                         lambda t, te_r, na_r: (te_r[t], 0, 0)),
        ],
        out_specs=pl.BlockSpec((F_TILE, C), lambda t, te_r, na_r: (t, 0)),
    )
    ys = pl.pallas_call(
        _ffn_body,
        grid_spec=grid_spec,
        out_shape=jax.ShapeDtypeStruct((MAX_ROWS, C), jnp.float32),
    )(te[0], nact.reshape((1,)), xs, W_fc, W_proj)

    tpw = N_TOKENS // 32
    combine = pl.kernel(
        _combine_body,
        mesh=mesh,
        out_type=jax.ShapeDtypeStruct((N, C), jnp.float32),
        scratch_types=[
            pltpu.VMEM((tpw,), jnp.int32),
            pltpu.VMEM((tpw,), jnp.int32),
            pltpu.VMEM((tpw,), jnp.float32),
            pltpu.VMEM((tpw,), jnp.float32),
            pltpu.VMEM((tpw, C), jnp.float32),
            pltpu.VMEM((tpw, C), jnp.float32),
            pltpu.SemaphoreType.DMA,
            pltpu.SemaphoreType.DMA,
            pltpu.SemaphoreType.DMA,
            pltpu.SemaphoreType.DMA,
            pltpu.SemaphoreType.DMA,
        ],
    )
    y_flat = combine(ys, pos, w)

    return (y_flat.reshape(B, T, C), laux[0, 0], zloss[0, 0])


# speedup vs baseline: 1.2228x; 1.0614x over previous
import functools

import jax
import jax.numpy as jnp
from jax import lax
from jax.experimental import pallas as pl
from jax.experimental.pallas import tpu as pltpu
from jax.experimental.pallas import tpu_sc as plsc

N_EXPERTS = 8
TOP_K = 2
N_EMBD = 768
HIDDEN = 2048
N_TOKENS = 2048
T_TILE = 128
N_GRP = N_TOKENS // T_TILE
N_ENTRY_ROWS = TOP_K * N_GRP
F_TILE = 576
MAX_TILES = 16
MAX_ROWS = MAX_TILES * F_TILE


def _router_meta_body(x_ref, rw_ref, pos_ref, w_ref, te_ref, nact_ref,
                      laux_ref, zloss_ref):
    x = x_ref[...]
    rw = rw_ref[...]
    logits = jnp.dot(x, rw, preferred_element_type=jnp.float32)
    m = jnp.max(logits, axis=-1, keepdims=True)
    ex = jnp.exp(logits - m)
    se = jnp.sum(ex, axis=-1, keepdims=True)
    probs = ex / se

    cols = lax.broadcasted_iota(jnp.int32, probs.shape, 1)
    w1 = jnp.max(probs, axis=-1, keepdims=True)
    i1 = jnp.argmax(probs, axis=-1)
    is1 = cols == i1[:, None]
    probs2 = jnp.where(is1, -jnp.inf, probs)
    w2 = jnp.max(probs2, axis=-1, keepdims=True)
    i2 = jnp.argmax(probs2, axis=-1)
    denom = w1 + w2 + 1e-9

    n = jnp.float32(N_TOKENS)
    load = jnp.sum(is1.astype(jnp.float32), axis=0) / n
    importance = jnp.mean(probs, axis=0)
    laux_ref[0, 0] = N_EXPERTS * jnp.sum(load * importance)
    lse = m[:, 0] + jnp.log(se[:, 0])
    zloss_ref[0, 0] = jnp.mean(lse * lse)

    e1r = jnp.reshape(i1, (N_GRP, T_TILE))
    e2r = jnp.reshape(i2, (N_GRP, T_TILE))
    w1r = jnp.reshape(w1[:, 0] / denom[:, 0], (N_GRP, T_TILE))
    w2r = jnp.reshape(w2[:, 0] / denom[:, 0], (N_GRP, T_TILE))
    er = jnp.concatenate([e1r, e2r], axis=0)
    w_ref[...] = jnp.concatenate([w1r, w2r], axis=0)

    ra = lax.broadcasted_iota(jnp.int32, (T_TILE, T_TILE), 0)
    ca = lax.broadcasted_iota(jnp.int32, (T_TILE, T_TILE), 1)
    U = (ra < ca).astype(jnp.float32)
    rg = lax.broadcasted_iota(jnp.int32, (N_ENTRY_ROWS, N_ENTRY_ROWS), 0)
    cg = lax.broadcasted_iota(jnp.int32, (N_ENTRY_ROWS, N_ENTRY_ROWS), 1)
    Lg = (rg > cg).astype(jnp.float32)

    inds, ranks, counts = [], [], []
    for e in range(N_EXPERTS):
        ind = (er == e).astype(jnp.float32)
        rank_in = jnp.dot(ind, U, preferred_element_type=jnp.float32)
        tot = jnp.sum(ind, axis=1, keepdims=True)
        grp = jnp.dot(Lg, tot, preferred_element_type=jnp.float32)
        inds.append(ind)
        ranks.append(rank_in + grp)
        counts.append(jnp.sum(tot, axis=0, keepdims=True))

    start = jnp.zeros((1, 1), jnp.int32)
    pos_f = jnp.zeros((N_ENTRY_ROWS, T_TILE), jnp.float32)
    tile_starts = []
    for e in range(N_EXPERTS):
        ci = counts[e].astype(jnp.int32)
        pc = ((ci + (F_TILE - 1)) // F_TILE) * F_TILE
        tile_starts.append(start // F_TILE)
        pos_f = pos_f + inds[e] * (start.astype(jnp.float32) + ranks[e])
        start = start + pc
    pos_ref[...] = pos_f.astype(jnp.int32)
    nact_ref[0, 0] = (start // F_TILE)[0, 0]

    tio = lax.broadcasted_iota(jnp.int32, (1, T_TILE), 1)
    te = jnp.zeros((1, T_TILE), jnp.int32)
    for e in range(N_EXPERTS):
        te = te + (tio >= tile_starts[e]).astype(jnp.int32)
    te_ref[...] = te - 1


def _dispatch_body(x_hbm, pos_hbm, xs_hbm, idx_v, rows_v, semi, semx, sems):
    wid = lax.axis_index("s") * 2 + lax.axis_index("c")
    tok0 = (wid % N_GRP) * T_TILE
    hh = T_TILE // 2
    ci0 = pltpu.async_copy(pos_hbm.at[wid, pl.ds(0, hh)], idx_v.at[0], semi)
    ci1 = pltpu.async_copy(pos_hbm.at[wid, pl.ds(hh, hh)], idx_v.at[1], semi)
    c0 = pltpu.async_copy(x_hbm.at[pl.ds(tok0, hh)], rows_v.at[pl.ds(0, hh)],
                          semx)
    c1 = pltpu.async_copy(x_hbm.at[pl.ds(tok0 + hh, hh)],
                          rows_v.at[pl.ds(hh, hh)], semx)
    ci0.wait()
    ci1.wait()
    c0.wait()
    s0 = pltpu.async_copy(rows_v.at[pl.ds(0, hh)], xs_hbm.at[idx_v.at[0]],
                          sems)
    c1.wait()
    s1 = pltpu.async_copy(rows_v.at[pl.ds(hh, hh)], xs_hbm.at[idx_v.at[1]],
                          sems)
    s0.wait()
    s1.wait()


def _ffn_body(te_ref, nact_ref, xs_ref, wfc_ref, wproj_ref, y_ref):
    t = pl.program_id(0)

    @pl.when(t < nact_ref[0])
    def _():
        x = xs_ref[...]
        h = jnp.dot(x, wfc_ref[0], preferred_element_type=jnp.float32)
        gate = h[:, :HIDDEN]
        val = h[:, HIDDEN:]
        act = gate * jax.nn.sigmoid(gate) * val
        y_ref[...] = jnp.dot(act, wproj_ref[0],
                             preferred_element_type=jnp.float32)


def _combine_body(ys_hbm, pos_hbm, w_hbm, y_hbm,
                  idx1_v, idx2_v, w1_v, w2_v, r1_v, r2_v,
                  sem1, sem2, sem3, sem4, sem5):
    wid = lax.axis_index("s") * 2 + lax.axis_index("c")
    tpw = N_TOKENS // 32
    hh = tpw // 2
    g = wid // 2
    off = (wid % 2) * tpw
    pltpu.sync_copy(pos_hbm.at[g, pl.ds(off, tpw)], idx1_v)
    pltpu.sync_copy(pos_hbm.at[N_GRP + g, pl.ds(off, tpw)], idx2_v)
    pltpu.sync_copy(w_hbm.at[g, pl.ds(off, tpw)], w1_v)
    pltpu.sync_copy(w_hbm.at[N_GRP + g, pl.ds(off, tpw)], w2_v)
    g1a = pltpu.async_copy(ys_hbm.at[idx1_v.at[pl.ds(0, hh)]],
                           r1_v.at[pl.ds(0, hh)], sem1)
    g2a = pltpu.async_copy(ys_hbm.at[idx2_v.at[pl.ds(0, hh)]],
                           r2_v.at[pl.ds(0, hh)], sem2)
    g1b = pltpu.async_copy(ys_hbm.at[idx1_v.at[pl.ds(hh, hh)]],
                           r1_v.at[pl.ds(hh, hh)], sem3)
    g2b = pltpu.async_copy(ys_hbm.at[idx2_v.at[pl.ds(hh, hh)]],
                           r2_v.at[pl.ds(hh, hh)], sem4)

    dnums = lax.GatherDimensionNumbers(
        offset_dims=(), collapsed_slice_dims=(0,), start_index_map=(0,))

    def mult_quarter(jc):
        wch1 = w1_v[pl.ds(jc * 16, 16)]
        wch2 = w2_v[pl.ds(jc * 16, 16)]

        def body(l, _):
            idx = jnp.full((16, 1), l, jnp.int32)
            w1b = lax.gather(wch1, idx, dnums, (1,),
                             mode=lax.GatherScatterMode.PROMISE_IN_BOUNDS)
            w2b = lax.gather(wch2, idx, dnums, (1,),
                             mode=lax.GatherScatterMode.PROMISE_IN_BOUNDS)
            j = jc * 16 + l
            for cc in range(N_EMBD // 16):
                sl = pl.ds(cc * 16, 16)
                r1_v[j, sl] = r1_v[j, sl] * w1b + r2_v[j, sl] * w2b
            return 0

        lax.fori_loop(0, 16, body, 0)

    g1a.wait()
    g2a.wait()
    mult_quarter(0)
    mult_quarter(1)
    st0 = pltpu.async_copy(r1_v.at[pl.ds(0, hh)],
                           y_hbm.at[pl.ds(wid * tpw, hh)], sem5)
    g1b.wait()
    g2b.wait()
    mult_quarter(2)
    mult_quarter(3)
    st0.wait()
    pltpu.sync_copy(r1_v.at[pl.ds(hh, hh)],
                    y_hbm.at[pl.ds(wid * tpw + hh, hh)])


def kernel(x, router_W, W_fc, W_proj):
    B, T, C = x.shape
    N = B * T
    x_flat = x.reshape(N, C)

    pos, w, te, nact, laux, zloss = pl.pallas_call(
        _router_meta_body,
        out_shape=(
            jax.ShapeDtypeStruct((N_ENTRY_ROWS, T_TILE), jnp.int32),
            jax.ShapeDtypeStruct((N_ENTRY_ROWS, T_TILE), jnp.float32),
            jax.ShapeDtypeStruct((1, T_TILE), jnp.int32),
            jax.ShapeDtypeStruct((1, 1), jnp.int32),
            jax.ShapeDtypeStruct((1, 1), jnp.float32),
            jax.ShapeDtypeStruct((1, 1), jnp.float32),
        ),
        in_specs=[
            pl.BlockSpec(memory_space=pltpu.VMEM),
            pl.BlockSpec(memory_space=pltpu.VMEM),
        ],
        out_specs=(
            pl.BlockSpec(memory_space=pltpu.VMEM),
            pl.BlockSpec(memory_space=pltpu.VMEM),
            pl.BlockSpec(memory_space=pltpu.VMEM),
            pl.BlockSpec(memory_space=pltpu.SMEM),
            pl.BlockSpec(memory_space=pltpu.SMEM),
            pl.BlockSpec(memory_space=pltpu.SMEM),
        ),
    )(x_flat, router_W)

    mesh = plsc.VectorSubcoreMesh(core_axis_name="c", subcore_axis_name="s")

    dispatch = pl.kernel(
        _dispatch_body,
        mesh=mesh,
        out_type=jax.ShapeDtypeStruct((MAX_ROWS, C), jnp.float32),
        scratch_types=[
            pltpu.VMEM((2, T_TILE // 2), jnp.int32),
            pltpu.VMEM((T_TILE, C), jnp.float32),
            pltpu.SemaphoreType.DMA,
            pltpu.SemaphoreType.DMA,
            pltpu.SemaphoreType.DMA,
        ],
    )
    xs = dispatch(x_flat, pos)

    grid_spec = pltpu.PrefetchScalarGridSpec(
        num_scalar_prefetch=2,
        grid=(MAX_TILES,),
        in_specs=[
            pl.BlockSpec((F_TILE, C), lambda t, te_r, na_r: (t, 0)),
            pl.BlockSpec((1, C, 2 * HIDDEN),
                         lambda t, te_r, na_r: (te_r[t], 0, 0)),
            pl.BlockSpec((1, HIDDEN, C),
                         lambda t, te_r, na_r: (te_r[t], 0, 0)),
        ],
        out_specs=pl.BlockSpec((F_TILE, C), lambda t, te_r, na_r: (t, 0)),
    )
    ys = pl.pallas_call(
        _ffn_body,
        grid_spec=grid_spec,
        out_shape=jax.ShapeDtypeStruct((MAX_ROWS, C), jnp.float32),
    )(te[0], nact.reshape((1,)), xs, W_fc, W_proj)

    tpw = N_TOKENS // 32
    combine = pl.kernel(
        _combine_body,
        mesh=mesh,
        out_type=jax.ShapeDtypeStruct((N, C), jnp.float32),
        scratch_types=[
            pltpu.VMEM((tpw,), jnp.int32),
            pltpu.VMEM((tpw,), jnp.int32),
            pltpu.VMEM((tpw,), jnp.float32),
            pltpu.VMEM((tpw,), jnp.float32),
            pltpu.VMEM((tpw, C), jnp.float32),
            pltpu.VMEM((tpw, C), jnp.float32),
            pltpu.SemaphoreType.DMA,
            pltpu.SemaphoreType.DMA,
            pltpu.SemaphoreType.DMA,
            pltpu.SemaphoreType.DMA,
            pltpu.SemaphoreType.DMA,
        ],
    )
    y_flat = combine(ys, pos, w)

    return (y_flat.reshape(B, T, C), laux[0, 0], zloss[0, 0])
